# baseline (speedup 1.0000x reference)
import functools

import jax
import jax.numpy as jnp
from jax import lax
from jax.experimental import pallas as pl
from jax.experimental.pallas import tpu as pltpu
from jax.experimental.pallas import tpu_sc as plsc

B, S, H = 4, 2048, 1024
T = 2
N = B * S
NW = 32
TPW = N // NW
C = 32
NCHUNK = TPW // C
NBLK = 256

_mesh = plsc.VectorSubcoreMesh(core_axis_name="c", subcore_axis_name="s")


@functools.partial(
    pl.kernel,
    mesh=_mesh,
    out_type=jax.ShapeDtypeStruct((N, H), jnp.float32),
    scratch_types=[
        pltpu.VMEM((TPW,), jnp.int32),
        pltpu.VMEM((2, C, H), jnp.float32),
        pltpu.SemaphoreType.DMA,
        pltpu.SemaphoreType.DMA,
        pltpu.SemaphoreType.DMA,
        pltpu.SemaphoreType.DMA,
    ],
)
def _sc_gather(ids_hbm, word_hbm, out_hbm, idx_v, wbuf, g0, g1, o0, o1):
    wid = lax.axis_index("s") * 2 + lax.axis_index("c")
    tok0 = wid * TPW
    gsem = (g0, g1)
    osem = (o0, o1)

    pltpu.sync_copy(ids_hbm.at[pl.ds(tok0, TPW)], idx_v)

    def start_gather(k):
        idx = idx_v.at[pl.ds(k * C, C)]
        return pltpu.async_copy(word_hbm.at[idx], wbuf.at[k % 2], gsem[k % 2])

    gcp = {0: start_gather(0), 1: start_gather(1)}
    ocp = {}
    for k in range(NCHUNK):
        gcp.pop(k).wait()
        ocp[k] = pltpu.async_copy(wbuf.at[k % 2],
                                  out_hbm.at[pl.ds(tok0 + k * C, C)],
                                  osem[k % 2])
        if k + 2 < NCHUNK:
            ocp.pop(k).wait()
            gcp[k + 2] = start_gather(k + 2)
    for d in ocp.values():
        d.wait()


def _add_body(w_ref, pos_ref, typ_ref, tt_ref, out_ref):
    mask = tt_ref[0, :, :] == 0
    typed = jnp.where(mask, typ_ref[0:1, :], typ_ref[1:2, :])
    out_ref[...] = w_ref[...] + pos_ref[...] + typed


def _tc_add(w, pos, typ, tts):
    grid = (S // NBLK, B)
    tt3 = tts.reshape(N // NBLK, NBLK, 1)
    nj = S // NBLK
    return pl.pallas_call(
        _add_body,
        grid=grid,
        in_specs=[
            pl.BlockSpec((NBLK, H), lambda j, b: (b * nj + j, 0)),
            pl.BlockSpec((NBLK, H), lambda j, b: (j, 0)),
            pl.BlockSpec((T, H), lambda j, b: (0, 0)),
            pl.BlockSpec((1, NBLK, 1), lambda j, b: (b * nj + j, 0, 0)),
        ],
        out_specs=pl.BlockSpec((NBLK, H), lambda j, b: (b * nj + j, 0)),
        out_shape=jax.ShapeDtypeStruct((N, H), jnp.float32),
    )(w, pos, typ, tt3)


def kernel(input_ids, token_type_ids, word_embeddings, position_embeddings,
           token_type_embeddings):
    ids = input_ids.reshape(N).astype(jnp.int32)
    tts = token_type_ids.reshape(N).astype(jnp.int32)
    w = _sc_gather(ids, word_embeddings)
    out = _tc_add(w, position_embeddings, token_type_embeddings, tts)
    return out.reshape(B, S, H)


# speedup vs baseline: 1.1622x; 1.1622x over previous
import functools

import jax
import jax.numpy as jnp
from jax import lax
from jax.experimental import pallas as pl
from jax.experimental.pallas import tpu as pltpu
from jax.experimental.pallas import tpu_sc as plsc

B, S, H = 4, 2048, 1024
T = 2
N = B * S
NW = 32
TPW = N // NW
C = 32
NCHUNK = TPW // C
NBLK = 1024

_mesh = plsc.VectorSubcoreMesh(core_axis_name="c", subcore_axis_name="s")


@functools.partial(
    pl.kernel,
    mesh=_mesh,
    out_type=jax.ShapeDtypeStruct((N, H), jnp.float32),
    scratch_types=[
        pltpu.VMEM((TPW,), jnp.int32),
        pltpu.VMEM((2, C, H), jnp.float32),
        pltpu.SemaphoreType.DMA,
        pltpu.SemaphoreType.DMA,
        pltpu.SemaphoreType.DMA,
        pltpu.SemaphoreType.DMA,
    ],
)
def _sc_gather(ids_hbm, word_hbm, out_hbm, idx_v, wbuf, g0, g1, o0, o1):
    wid = lax.axis_index("s") * 2 + lax.axis_index("c")
    tok0 = wid * TPW
    gsem = (g0, g1)
    osem = (o0, o1)

    pltpu.sync_copy(ids_hbm.at[pl.ds(tok0, TPW)], idx_v)

    def start_gather(k):
        idx = idx_v.at[pl.ds(k * C, C)]
        return pltpu.async_copy(word_hbm.at[idx], wbuf.at[k % 2], gsem[k % 2])

    gcp = {0: start_gather(0), 1: start_gather(1)}
    ocp = {}
    for k in range(NCHUNK):
        gcp.pop(k).wait()
        ocp[k] = pltpu.async_copy(wbuf.at[k % 2],
                                  out_hbm.at[pl.ds(tok0 + k * C, C)],
                                  osem[k % 2])
        if k + 2 < NCHUNK:
            ocp.pop(k).wait()
            gcp[k + 2] = start_gather(k + 2)
    for d in ocp.values():
        d.wait()


def _add_body(w_ref, pos_ref, typ_ref, tt_ref, out_ref):
    mask = tt_ref[0, :, :] == 0
    typed = jnp.where(mask, typ_ref[0:1, :], typ_ref[1:2, :])
    out_ref[...] = w_ref[...] + pos_ref[...] + typed


def _tc_add(w, pos, typ, tts):
    grid = (S // NBLK, B)
    tt3 = tts.reshape(N // NBLK, NBLK, 1)
    nj = S // NBLK
    return pl.pallas_call(
        _add_body,
        grid=grid,
        in_specs=[
            pl.BlockSpec((NBLK, H), lambda j, b: (b * nj + j, 0)),
            pl.BlockSpec((NBLK, H), lambda j, b: (j, 0)),
            pl.BlockSpec((T, H), lambda j, b: (0, 0)),
            pl.BlockSpec((1, NBLK, 1), lambda j, b: (b * nj + j, 0, 0)),
        ],
        out_specs=pl.BlockSpec((NBLK, H), lambda j, b: (b * nj + j, 0)),
        out_shape=jax.ShapeDtypeStruct((N, H), jnp.float32),
    )(w, pos, typ, tt3)


def kernel(input_ids, token_type_ids, word_embeddings, position_embeddings,
           token_type_embeddings):
    ids = input_ids.reshape(N).astype(jnp.int32)
    tts = token_type_ids.reshape(N).astype(jnp.int32)
    w = _sc_gather(ids, word_embeddings)
    out = _tc_add(w, position_embeddings, token_type_embeddings, tts)
    return out.reshape(B, S, H)


# speedup vs baseline: 1.1869x; 1.0213x over previous
import functools

import jax
import jax.numpy as jnp
from jax import lax
from jax.experimental import pallas as pl
from jax.experimental.pallas import tpu as pltpu
from jax.experimental.pallas import tpu_sc as plsc

B, S, H = 4, 2048, 1024
T = 2
N = B * S
NW = 32
TPW = N // NW
C = 32
NCHUNK = TPW // C
NBLK = 2048

_mesh = plsc.VectorSubcoreMesh(core_axis_name="c", subcore_axis_name="s")


@functools.partial(
    pl.kernel,
    mesh=_mesh,
    out_type=jax.ShapeDtypeStruct((N, H), jnp.float32),
    scratch_types=[
        pltpu.VMEM((TPW,), jnp.int32),
        pltpu.VMEM((2, C, H), jnp.float32),
        pltpu.SemaphoreType.DMA,
        pltpu.SemaphoreType.DMA,
        pltpu.SemaphoreType.DMA,
        pltpu.SemaphoreType.DMA,
    ],
)
def _sc_gather(ids_hbm, word_hbm, out_hbm, idx_v, wbuf, g0, g1, o0, o1):
    wid = lax.axis_index("s") * 2 + lax.axis_index("c")
    tok0 = wid * TPW
    gsem = (g0, g1)
    osem = (o0, o1)

    pltpu.sync_copy(ids_hbm.at[pl.ds(tok0, TPW)], idx_v)

    def start_gather(k):
        idx = idx_v.at[pl.ds(k * C, C)]
        return pltpu.async_copy(word_hbm.at[idx], wbuf.at[k % 2], gsem[k % 2])

    gcp = {0: start_gather(0), 1: start_gather(1)}
    ocp = {}
    for k in range(NCHUNK):
        gcp.pop(k).wait()
        ocp[k] = pltpu.async_copy(wbuf.at[k % 2],
                                  out_hbm.at[pl.ds(tok0 + k * C, C)],
                                  osem[k % 2])
        if k + 2 < NCHUNK:
            ocp.pop(k).wait()
            gcp[k + 2] = start_gather(k + 2)
    for d in ocp.values():
        d.wait()


def _add_body(w_ref, pos_ref, typ_ref, tt_ref, out_ref):
    mask = tt_ref[0, :, :] == 0
    typed = jnp.where(mask, typ_ref[0:1, :], typ_ref[1:2, :])
    out_ref[...] = w_ref[...] + pos_ref[...] + typed


def _tc_add(w, pos, typ, tts):
    grid = (S // NBLK, B)
    tt3 = tts.reshape(N // NBLK, NBLK, 1)
    nj = S // NBLK
    return pl.pallas_call(
        _add_body,
        grid=grid,
        in_specs=[
            pl.BlockSpec((NBLK, H), lambda j, b: (b * nj + j, 0)),
            pl.BlockSpec((NBLK, H), lambda j, b: (j, 0)),
            pl.BlockSpec((T, H), lambda j, b: (0, 0)),
            pl.BlockSpec((1, NBLK, 1), lambda j, b: (b * nj + j, 0, 0)),
        ],
        out_specs=pl.BlockSpec((NBLK, H), lambda j, b: (b * nj + j, 0)),
        out_shape=jax.ShapeDtypeStruct((N, H), jnp.float32),
    )(w, pos, typ, tt3)


def kernel(input_ids, token_type_ids, word_embeddings, position_embeddings,
           token_type_embeddings):
    ids = input_ids.reshape(N).astype(jnp.int32)
    tts = token_type_ids.reshape(N).astype(jnp.int32)
    w = _sc_gather(ids, word_embeddings)
    out = _tc_add(w, position_embeddings, token_type_embeddings, tts)
    return out.reshape(B, S, H)
